# baseline (speedup 1.0000x reference)
import functools

import jax
import jax.numpy as jnp
from jax import lax
from jax.experimental import pallas as pl
from jax.experimental.pallas import tpu as pltpu
from jax.experimental.pallas import tpu_sc as plsc

N = 10000
E = 320000
D = 128
H = 128
OUT = 64
G = 128
L = 4
EPS = 1e-5

NC = 2
NS = 16
NW = NC * NS
EW = E // NW
CSZ = 80
CHUNKS = EW // CSZ
NPAD = 10000
RPT = NPAD // NS
ZROWS = 25
RING = 4


def _agg_body(h_hbm, sd_hbm, out_hbm, idxb, rows, acc, isem, gsem, ssem):
    c = lax.axis_index("c")
    s = lax.axis_index("s")
    wid = s * NC + c

    def _zb(k, carry):
        rows[0, k // 8, pl.ds((k % 8) * 16, 16)] = jnp.zeros((16,), jnp.float32)
        return carry
    lax.fori_loop(0, ZROWS * 8, _zb, None)

    def _zc(t, carry):
        pltpu.sync_copy(rows.at[0, pl.ds(0, ZROWS)],
                        acc.at[pl.ds(s * RPT + t * ZROWS, ZROWS)])
        return carry
    lax.fori_loop(0, RPT // ZROWS, _zc, None)
    plsc.subcore_barrier()

    pltpu.async_copy(sd_hbm.at[wid, 0], idxb.at[0], isem.at[0])
    pltpu.async_copy(sd_hbm.at[wid, 1], idxb.at[1], isem.at[1])

    def _body(j, carry):
        sl = lax.rem(j, RING)
        pltpu.make_async_copy(sd_hbm.at[wid, 0], idxb.at[sl],
                              isem.at[sl]).wait()

        @pl.when(j >= 2)
        def _():
            sl2 = lax.rem(j - 2, RING)
            pltpu.make_async_copy(rows.at[sl2], acc.at[idxb.at[sl2, 1]],
                                  ssem.at[sl2]).wait()

        pltpu.async_copy(h_hbm.at[idxb.at[sl, 0]], rows.at[sl], gsem.at[sl])

        @pl.when(j + 2 < CHUNKS)
        def _():
            sl3 = lax.rem(j + 2, RING)
            pltpu.async_copy(sd_hbm.at[wid, j + 2], idxb.at[sl3],
                             isem.at[sl3])

        pltpu.make_async_copy(h_hbm.at[idxb.at[sl, 0]], rows.at[sl],
                              gsem.at[sl]).wait()
        pltpu.async_copy(rows.at[sl], acc.at[idxb.at[sl, 1]], ssem.at[sl],
                         add=True)
        return carry
    lax.fori_loop(0, CHUNKS, _body, None)
    for j in (CHUNKS - 2, CHUNKS - 1):
        sl = j % RING
        pltpu.make_async_copy(rows.at[sl], acc.at[idxb.at[sl, 1]],
                              ssem.at[sl]).wait()

    plsc.subcore_barrier()
    pltpu.sync_copy(acc.at[pl.ds(s * RPT, RPT)],
                    out_hbm.at[c, pl.ds(s * RPT, RPT)])


def _edge_agg(h, sd_r):
    mesh = plsc.VectorSubcoreMesh(core_axis_name="c", subcore_axis_name="s")
    fn = pl.kernel(
        _agg_body,
        out_type=jax.ShapeDtypeStruct((NC, NPAD, H), jnp.float32),
        mesh=mesh,
        scratch_types=[
            pltpu.VMEM((RING, 2, CSZ), jnp.int32),
            pltpu.VMEM((RING, CSZ, H), jnp.float32),
            pltpu.VMEM_SHARED((NPAD, H), jnp.float32),
            pltpu.SemaphoreType.DMA((RING,)),
            pltpu.SemaphoreType.DMA((RING,)),
            pltpu.SemaphoreType.DMA((RING,)),
        ],
        compiler_params=pltpu.CompilerParams(use_tc_tiling_on_sc=False),
        name="gin_edge_agg",
    )
    return fn(h, sd_r)


def _bn(a, g, b):
    mu = jnp.mean(a, axis=0, keepdims=True)
    d = a - mu
    var = jnp.mean(d * d, axis=0, keepdims=True)
    return g * (d * lax.rsqrt(var + EPS)) + b


def _mlp_core(h_ref, p_ref, w1, b1, g1, be1, w2, b2, g2, be2):
    h = h_ref[...]
    m = h + p_ref[0, :N, :] + p_ref[1, :N, :]
    a = jnp.dot(m, w1[...], preferred_element_type=jnp.float32) + b1[...]
    a = jnp.maximum(_bn(a, g1[...], be1[...]), 0.0)
    a = jnp.dot(a, w2[...], preferred_element_type=jnp.float32) + b2[...]
    a = jnp.maximum(_bn(a, g2[...], be2[...]), 0.0)
    return h, a


def _mlp_first_body(h_ref, p_ref, w1, b1, g1, be1, w2, b2, g2, be2,
                    wfc0, wfc1, hout_ref, zout_ref):
    h, a = _mlp_core(h_ref, p_ref, w1, b1, g1, be1, w2, b2, g2, be2)
    hout_ref[...] = a
    zout_ref[...] = (jnp.dot(h, wfc0[...], preferred_element_type=jnp.float32)
                     + jnp.dot(a, wfc1[...], preferred_element_type=jnp.float32))


def _mlp_mid_body(h_ref, p_ref, w1, b1, g1, be1, w2, b2, g2, be2,
                  wfc, zin_ref, hout_ref, zout_ref):
    _, a = _mlp_core(h_ref, p_ref, w1, b1, g1, be1, w2, b2, g2, be2)
    hout_ref[...] = a
    zout_ref[...] = zin_ref[...] + jnp.dot(
        a, wfc[...], preferred_element_type=jnp.float32)


def _mlp_last_body(h_ref, p_ref, w1, b1, g1, be1, w2, b2, g2, be2,
                   wfc, zin_ref, batch_ref, bias_ref, out_ref):
    _, a = _mlp_core(h_ref, p_ref, w1, b1, g1, be1, w2, b2, g2, be2)
    z = zin_ref[...] + jnp.dot(a, wfc[...], preferred_element_type=jnp.float32)
    row = lax.broadcasted_iota(jnp.int32, (G, N), 0)
    sel = (row == jnp.broadcast_to(batch_ref[...], (G, N))).astype(jnp.float32)
    bias = jnp.sum(bias_ref[...], axis=0, keepdims=True)
    out_ref[...] = jnp.dot(sel, z, preferred_element_type=jnp.float32) + bias


def kernel(x, edge_index, batch, params):
    src_r = edge_index[0].reshape(NW, CHUNKS, CSZ)
    dst_r = edge_index[1].reshape(NW, CHUNKS, CSZ)
    sd_r = jnp.stack([src_r, dst_r], axis=2)
    batch2 = batch.reshape(1, N)
    fcs = params['fcs']
    bias_stack = jnp.stack([fcs[i]['b'] for i in range(L + 1)])

    def cp(i):
        p = params['conv%d' % i]
        return (p['W1'], p['b1'].reshape(1, H), p['g1'].reshape(1, H),
                p['be1'].reshape(1, H), p['W2'], p['b2'].reshape(1, H),
                p['g'].reshape(1, H), p['be'].reshape(1, H))

    hz_shape = [jax.ShapeDtypeStruct((N, H), jnp.float32),
                jax.ShapeDtypeStruct((N, OUT), jnp.float32)]

    p = _edge_agg(x, sd_r)
    h, z = pl.pallas_call(_mlp_first_body, out_shape=hz_shape)(
        x, p, *cp(0), fcs[0]['W'], fcs[1]['W'])

    for i in (1, 2):
        p = _edge_agg(h, sd_r)
        h, z = pl.pallas_call(_mlp_mid_body, out_shape=hz_shape)(
            h, p, *cp(i), fcs[i + 1]['W'], z)

    p = _edge_agg(h, sd_r)
    out = pl.pallas_call(
        _mlp_last_body,
        out_shape=jax.ShapeDtypeStruct((G, OUT), jnp.float32))(
        h, p, *cp(3), fcs[4]['W'], z, batch2, bias_stack)
    return out


# speedup vs baseline: 1.1061x; 1.1061x over previous
import functools

import jax
import jax.numpy as jnp
from jax import lax
from jax.experimental import pallas as pl
from jax.experimental.pallas import tpu as pltpu
from jax.experimental.pallas import tpu_sc as plsc

N = 10000
E = 320000
D = 128
H = 128
OUT = 64
G = 128
L = 4
EPS = 1e-5

NC = 2
NS = 16
NW = NC * NS
EW = E // NW
CSZ = 100
CHUNKS = EW // CSZ
NPAD = 10240
RPT = NPAD // NS
ZROWS = 80


def _agg_body(h_hbm, src_hbm, dst_hbm, out_hbm,
              idx_s, idx_d, rows0, rows1, acc, sem0, sem1, ssem0, ssem1):
    c = lax.axis_index("c")
    s = lax.axis_index("s")
    wid = s * NC + c

    def _zb(k, carry):
        rows0[k // 8, pl.ds((k % 8) * 16, 16)] = jnp.zeros((16,), jnp.float32)
        return carry
    lax.fori_loop(0, ZROWS * 8, _zb, None)

    def _zc(t, carry):
        pltpu.sync_copy(rows0.at[pl.ds(0, ZROWS)],
                        acc.at[pl.ds(s * RPT + t * ZROWS, ZROWS)])
        return carry
    lax.fori_loop(0, RPT // ZROWS, _zc, None)
    plsc.subcore_barrier()

    pltpu.sync_copy(src_hbm.at[wid], idx_s)
    pltpu.sync_copy(dst_hbm.at[wid], idx_d)

    pltpu.async_copy(h_hbm.at[idx_s.at[0]], rows0, sem0)

    def _body(jj, carry):
        j0 = 2 * jj
        j1 = j0 + 1
        pltpu.make_async_copy(h_hbm.at[idx_s.at[j0]], rows0, sem0).wait()
        pltpu.async_copy(rows0, acc.at[idx_d.at[j0]], ssem0, add=True)

        @pl.when(jj >= 1)
        def _():
            pltpu.make_async_copy(rows1, acc.at[idx_d.at[j0]], ssem1).wait()

        pltpu.async_copy(h_hbm.at[idx_s.at[j1]], rows1, sem1)
        pltpu.make_async_copy(h_hbm.at[idx_s.at[j1]], rows1, sem1).wait()
        pltpu.async_copy(rows1, acc.at[idx_d.at[j1]], ssem1, add=True)
        pltpu.make_async_copy(rows0, acc.at[idx_d.at[j1]], ssem0).wait()

        @pl.when(jj + 1 < CHUNKS // 2)
        def _():
            pltpu.async_copy(h_hbm.at[idx_s.at[j1 + 1]], rows0, sem0)

        return carry
    lax.fori_loop(0, CHUNKS // 2, _body, None)
    pltpu.make_async_copy(rows1, acc.at[idx_d.at[0]], ssem1).wait()

    plsc.subcore_barrier()
    pltpu.sync_copy(acc.at[pl.ds(s * RPT, RPT)],
                    out_hbm.at[c, pl.ds(s * RPT, RPT)])


def _edge_agg(h, src_r, dst_r):
    mesh = plsc.VectorSubcoreMesh(core_axis_name="c", subcore_axis_name="s")
    fn = pl.kernel(
        _agg_body,
        out_type=jax.ShapeDtypeStruct((NC, NPAD, H), jnp.float32),
        mesh=mesh,
        scratch_types=[
            pltpu.VMEM((CHUNKS, CSZ), jnp.int32),
            pltpu.VMEM((CHUNKS, CSZ), jnp.int32),
            pltpu.VMEM((CSZ, H), jnp.float32),
            pltpu.VMEM((CSZ, H), jnp.float32),
            pltpu.VMEM_SHARED((NPAD, H), jnp.float32),
            pltpu.SemaphoreType.DMA,
            pltpu.SemaphoreType.DMA,
            pltpu.SemaphoreType.DMA,
            pltpu.SemaphoreType.DMA,
        ],
        compiler_params=pltpu.CompilerParams(use_tc_tiling_on_sc=False),
        name="gin_edge_agg",
    )
    return fn(h, src_r, dst_r)


def _bn(a, g, b):
    mu = jnp.mean(a, axis=0, keepdims=True)
    d = a - mu
    var = jnp.mean(d * d, axis=0, keepdims=True)
    return g * (d * lax.rsqrt(var + EPS)) + b


def _mlp_core(h_ref, p_ref, w1, b1, g1, be1, w2, b2, g2, be2):
    h = h_ref[...]
    m = h + p_ref[0, :N, :] + p_ref[1, :N, :]
    a = jnp.dot(m, w1[...], preferred_element_type=jnp.float32) + b1[...]
    a = jnp.maximum(_bn(a, g1[...], be1[...]), 0.0)
    a = jnp.dot(a, w2[...], preferred_element_type=jnp.float32) + b2[...]
    a = jnp.maximum(_bn(a, g2[...], be2[...]), 0.0)
    return h, a


def _mlp_first_body(h_ref, p_ref, w1, b1, g1, be1, w2, b2, g2, be2,
                    wfc0, wfc1, hout_ref, zout_ref):
    h, a = _mlp_core(h_ref, p_ref, w1, b1, g1, be1, w2, b2, g2, be2)
    hout_ref[...] = a
    zout_ref[...] = (jnp.dot(h, wfc0[...], preferred_element_type=jnp.float32)
                     + jnp.dot(a, wfc1[...], preferred_element_type=jnp.float32))


def _mlp_mid_body(h_ref, p_ref, w1, b1, g1, be1, w2, b2, g2, be2,
                  wfc, zin_ref, hout_ref, zout_ref):
    _, a = _mlp_core(h_ref, p_ref, w1, b1, g1, be1, w2, b2, g2, be2)
    hout_ref[...] = a
    zout_ref[...] = zin_ref[...] + jnp.dot(
        a, wfc[...], preferred_element_type=jnp.float32)


def _mlp_last_body(h_ref, p_ref, w1, b1, g1, be1, w2, b2, g2, be2,
                   wfc, zin_ref, batch_ref, bias_ref, out_ref):
    _, a = _mlp_core(h_ref, p_ref, w1, b1, g1, be1, w2, b2, g2, be2)
    z = zin_ref[...] + jnp.dot(a, wfc[...], preferred_element_type=jnp.float32)
    row = lax.broadcasted_iota(jnp.int32, (G, N), 0)
    sel = (row == jnp.broadcast_to(batch_ref[...], (G, N))).astype(jnp.float32)
    bias = jnp.sum(bias_ref[...], axis=0, keepdims=True)
    out_ref[...] = jnp.dot(sel, z, preferred_element_type=jnp.float32) + bias


def kernel(x, edge_index, batch, params):
    src_r = edge_index[0].reshape(NW, CHUNKS, CSZ)
    dst_r = edge_index[1].reshape(NW, CHUNKS, CSZ)
    batch2 = batch.reshape(1, N)
    fcs = params['fcs']
    bias_stack = jnp.stack([fcs[i]['b'] for i in range(L + 1)])

    def cp(i):
        p = params['conv%d' % i]
        return (p['W1'], p['b1'].reshape(1, H), p['g1'].reshape(1, H),
                p['be1'].reshape(1, H), p['W2'], p['b2'].reshape(1, H),
                p['g'].reshape(1, H), p['be'].reshape(1, H))

    hz_shape = [jax.ShapeDtypeStruct((N, H), jnp.float32),
                jax.ShapeDtypeStruct((N, OUT), jnp.float32)]

    p = _edge_agg(x, src_r, dst_r)
    h, z = pl.pallas_call(_mlp_first_body, out_shape=hz_shape)(
        x, p, *cp(0), fcs[0]['W'], fcs[1]['W'])

    for i in (1, 2):
        p = _edge_agg(h, src_r, dst_r)
        h, z = pl.pallas_call(_mlp_mid_body, out_shape=hz_shape)(
            h, p, *cp(i), fcs[i + 1]['W'], z)

    p = _edge_agg(h, src_r, dst_r)
    out = pl.pallas_call(
        _mlp_last_body,
        out_shape=jax.ShapeDtypeStruct((G, OUT), jnp.float32))(
        h, p, *cp(3), fcs[4]['W'], z, batch2, bias_stack)
    return out


# speedup vs baseline: 1.1783x; 1.0653x over previous
import functools

import jax
import jax.numpy as jnp
from jax import lax
from jax.experimental import pallas as pl
from jax.experimental.pallas import tpu as pltpu
from jax.experimental.pallas import tpu_sc as plsc

N = 10000
E = 320000
D = 128
H = 128
OUT = 64
G = 128
L = 4
EPS = 1e-5

NC = 2
NS = 16
NW = NC * NS
EW = E // NW
CSZ = 50
CHUNKS = EW // CSZ
NPAD = 10000
RPT = NPAD // NS
ZROWS = 25


def _agg_body(h_hbm, src_hbm, dst_hbm, out_hbm,
              idx_s, idx_d, rows, acc, gsem, ssem):
    c = lax.axis_index("c")
    s = lax.axis_index("s")
    wid = s * NC + c

    def _zb(k, carry):
        rows[0, k // 8, pl.ds((k % 8) * 16, 16)] = jnp.zeros((16,), jnp.float32)
        return carry
    lax.fori_loop(0, ZROWS * 8, _zb, None)

    def _zc(t, carry):
        pltpu.sync_copy(rows.at[0, pl.ds(0, ZROWS)],
                        acc.at[pl.ds(s * RPT + t * ZROWS, ZROWS)])
        return carry
    lax.fori_loop(0, RPT // ZROWS, _zc, None)
    plsc.subcore_barrier()

    pltpu.sync_copy(src_hbm.at[wid], idx_s)
    pltpu.sync_copy(dst_hbm.at[wid], idx_d)

    def _g(j, b):
        pltpu.async_copy(h_hbm.at[idx_s.at[j]], rows.at[b], gsem.at[b])

    def _gw(j, b):
        pltpu.make_async_copy(h_hbm.at[idx_s.at[j]], rows.at[b],
                              gsem.at[b]).wait()

    def _s(j, b):
        pltpu.async_copy(rows.at[b], acc.at[idx_d.at[j]], ssem.at[b],
                         add=True)

    def _sw(j, b):
        pltpu.make_async_copy(rows.at[b], acc.at[idx_d.at[j]],
                              ssem.at[b]).wait()

    _g(0, 0)
    _g(1, 1)
    _gw(0, 0); _s(0, 0); _g(2, 2)
    _gw(1, 1); _s(1, 1); _g(3, 3)
    _gw(2, 2); _s(2, 2); _sw(0, 0); _g(4, 0)
    _gw(3, 3); _s(3, 3); _sw(1, 1); _g(5, 1)

    def _body(q, carry):
        j = 4 * q
        for k in range(4):
            _gw(j + k, k)
            _s(j + k, k)
            _sw(j + k - 2, (k + 2) % 4)
            _g(j + k + 2, (k + 2) % 4)
        return carry
    lax.fori_loop(1, CHUNKS // 4 - 1, _body, None)

    jt = CHUNKS - 4
    _gw(jt, 0); _s(jt, 0); _sw(jt - 2, 2); _g(jt + 2, 2)
    _gw(jt + 1, 1); _s(jt + 1, 1); _sw(jt - 1, 3); _g(jt + 3, 3)
    _gw(jt + 2, 2); _s(jt + 2, 2); _sw(jt, 0)
    _gw(jt + 3, 3); _s(jt + 3, 3); _sw(jt + 1, 1)
    _sw(jt + 2, 2)
    _sw(jt + 3, 3)

    plsc.subcore_barrier()
    pltpu.sync_copy(acc.at[pl.ds(s * RPT, RPT)],
                    out_hbm.at[c, pl.ds(s * RPT, RPT)])


def _edge_agg(h, src_r, dst_r):
    mesh = plsc.VectorSubcoreMesh(core_axis_name="c", subcore_axis_name="s")
    fn = pl.kernel(
        _agg_body,
        out_type=jax.ShapeDtypeStruct((NC, NPAD, H), jnp.float32),
        mesh=mesh,
        scratch_types=[
            pltpu.VMEM((CHUNKS, CSZ), jnp.int32),
            pltpu.VMEM((CHUNKS, CSZ), jnp.int32),
            pltpu.VMEM((4, CSZ, H), jnp.float32),
            pltpu.VMEM_SHARED((NPAD, H), jnp.float32),
            pltpu.SemaphoreType.DMA((4,)),
            pltpu.SemaphoreType.DMA((4,)),
        ],
        compiler_params=pltpu.CompilerParams(use_tc_tiling_on_sc=False),
        name="gin_edge_agg",
    )
    return fn(h, src_r, dst_r)


def _bn(a, g, b):
    mu = jnp.mean(a, axis=0, keepdims=True)
    d = a - mu
    var = jnp.mean(d * d, axis=0, keepdims=True)
    return g * (d * lax.rsqrt(var + EPS)) + b


def _mlp_core(h_ref, p_ref, w1, b1, g1, be1, w2, b2, g2, be2):
    h = h_ref[...]
    m = h + p_ref[0, :N, :] + p_ref[1, :N, :]
    a = jnp.dot(m, w1[...], preferred_element_type=jnp.float32) + b1[...]
    a = jnp.maximum(_bn(a, g1[...], be1[...]), 0.0)
    a = jnp.dot(a, w2[...], preferred_element_type=jnp.float32) + b2[...]
    a = jnp.maximum(_bn(a, g2[...], be2[...]), 0.0)
    return h, a


def _mlp_first_body(h_ref, p_ref, w1, b1, g1, be1, w2, b2, g2, be2,
                    wfc0, wfc1, hout_ref, zout_ref):
    h, a = _mlp_core(h_ref, p_ref, w1, b1, g1, be1, w2, b2, g2, be2)
    hout_ref[...] = a
    zout_ref[...] = (jnp.dot(h, wfc0[...], preferred_element_type=jnp.float32)
                     + jnp.dot(a, wfc1[...], preferred_element_type=jnp.float32))


def _mlp_mid_body(h_ref, p_ref, w1, b1, g1, be1, w2, b2, g2, be2,
                  wfc, zin_ref, hout_ref, zout_ref):
    _, a = _mlp_core(h_ref, p_ref, w1, b1, g1, be1, w2, b2, g2, be2)
    hout_ref[...] = a
    zout_ref[...] = zin_ref[...] + jnp.dot(
        a, wfc[...], preferred_element_type=jnp.float32)


def _mlp_last_body(h_ref, p_ref, w1, b1, g1, be1, w2, b2, g2, be2,
                   wfc, zin_ref, batch_ref, bias_ref, out_ref):
    _, a = _mlp_core(h_ref, p_ref, w1, b1, g1, be1, w2, b2, g2, be2)
    z = zin_ref[...] + jnp.dot(a, wfc[...], preferred_element_type=jnp.float32)
    row = lax.broadcasted_iota(jnp.int32, (G, N), 0)
    sel = (row == jnp.broadcast_to(batch_ref[...], (G, N))).astype(jnp.float32)
    bias = jnp.sum(bias_ref[...], axis=0, keepdims=True)
    out_ref[...] = jnp.dot(sel, z, preferred_element_type=jnp.float32) + bias


def kernel(x, edge_index, batch, params):
    src_r = edge_index[0].reshape(NW, CHUNKS, CSZ)
    dst_r = edge_index[1].reshape(NW, CHUNKS, CSZ)
    batch2 = batch.reshape(1, N)
    fcs = params['fcs']
    bias_stack = jnp.stack([fcs[i]['b'] for i in range(L + 1)])

    def cp(i):
        p = params['conv%d' % i]
        return (p['W1'], p['b1'].reshape(1, H), p['g1'].reshape(1, H),
                p['be1'].reshape(1, H), p['W2'], p['b2'].reshape(1, H),
                p['g'].reshape(1, H), p['be'].reshape(1, H))

    hz_shape = [jax.ShapeDtypeStruct((N, H), jnp.float32),
                jax.ShapeDtypeStruct((N, OUT), jnp.float32)]

    p = _edge_agg(x, src_r, dst_r)
    h, z = pl.pallas_call(_mlp_first_body, out_shape=hz_shape)(
        x, p, *cp(0), fcs[0]['W'], fcs[1]['W'])

    for i in (1, 2):
        p = _edge_agg(h, src_r, dst_r)
        h, z = pl.pallas_call(_mlp_mid_body, out_shape=hz_shape)(
            h, p, *cp(i), fcs[i + 1]['W'], z)

    p = _edge_agg(h, src_r, dst_r)
    out = pl.pallas_call(
        _mlp_last_body,
        out_shape=jax.ShapeDtypeStruct((G, OUT), jnp.float32))(
        h, p, *cp(3), fcs[4]['W'], z, batch2, bias_stack)
    return out


# speedup vs baseline: 1.4545x; 1.2344x over previous
import functools

import jax
import jax.numpy as jnp
from jax import lax
from jax.experimental import pallas as pl
from jax.experimental.pallas import tpu as pltpu
from jax.experimental.pallas import tpu_sc as plsc

N = 10000
E = 320000
D = 128
H = 128
OUT = 64
G = 128
L = 4
EPS = 1e-5

NC = 2
NS = 16
NW = NC * NS
EW = E // NW
CSZ = 100
CHUNKS = EW // CSZ
HALF = CHUNKS // 2
NPAD = 10000
RPT = NPAD // NS
ZROWS = 25


def _agg_body(h_hbm, src_hbm, dst_hbm, out_hbm,
              idx_s, idx_d, rows, acc, gsem, ssem):
    c = lax.axis_index("c")
    s = lax.axis_index("s")
    wid = s * NC + c

    def _zb(k, carry):
        rows[0, k // 8, pl.ds((k % 8) * 16, 16)] = jnp.zeros((16,), jnp.float32)
        return carry
    lax.fori_loop(0, ZROWS * 8, _zb, None)

    def _zc(t, carry):
        pltpu.sync_copy(rows.at[0, pl.ds(0, ZROWS)],
                        acc.at[pl.ds(s * RPT + t * ZROWS, ZROWS)])
        return carry
    lax.fori_loop(0, RPT // ZROWS, _zc, None)
    plsc.subcore_barrier()

    def _g(j, b):
        pltpu.async_copy(h_hbm.at[idx_s.at[j]], rows.at[b], gsem.at[b])

    def _gw(j, b):
        pltpu.make_async_copy(h_hbm.at[idx_s.at[j]], rows.at[b],
                              gsem.at[b]).wait()

    def _s(j, b):
        pltpu.async_copy(rows.at[b], acc.at[idx_d.at[j]], ssem.at[b],
                         add=True)

    def _sw(j, b):
        pltpu.make_async_copy(rows.at[b], acc.at[idx_d.at[j]],
                              ssem.at[b]).wait()

    for p in range(2):
        pltpu.sync_copy(src_hbm.at[wid, pl.ds(p * HALF, HALF)], idx_s)
        pltpu.sync_copy(dst_hbm.at[wid, pl.ds(p * HALF, HALF)], idx_d)

        _g(0, 0)
        _g(1, 1)
        _gw(0, 0); _s(0, 0); _g(2, 2)

        def _body(q, carry):
            for k in range(3):
                j = 3 * q + 1 + k
                b = (1 + k) % 3
                b2 = k % 3
                _gw(j, b)
                _s(j, b)
                _sw(j - 1, b2)
                _g(j + 2, b2)
            return carry
        lax.fori_loop(0, 15, _body, None)

        _gw(46, 1); _s(46, 1); _sw(45, 0); _g(48, 0)
        _gw(47, 2); _s(47, 2); _sw(46, 1); _g(49, 1)
        _gw(48, 0); _s(48, 0); _sw(47, 2)
        _gw(49, 1); _s(49, 1); _sw(48, 0)
        _sw(49, 1)

    plsc.subcore_barrier()
    pltpu.sync_copy(acc.at[pl.ds(s * RPT, RPT)],
                    out_hbm.at[c, pl.ds(s * RPT, RPT)])


def _edge_agg(h, src_r, dst_r):
    mesh = plsc.VectorSubcoreMesh(core_axis_name="c", subcore_axis_name="s")
    fn = pl.kernel(
        _agg_body,
        out_type=jax.ShapeDtypeStruct((NC, NPAD, H), jnp.float32),
        mesh=mesh,
        scratch_types=[
            pltpu.VMEM((HALF, CSZ), jnp.int32),
            pltpu.VMEM((HALF, CSZ), jnp.int32),
            pltpu.VMEM((3, CSZ, H), jnp.float32),
            pltpu.VMEM_SHARED((NPAD, H), jnp.float32),
            pltpu.SemaphoreType.DMA((3,)),
            pltpu.SemaphoreType.DMA((3,)),
        ],
        compiler_params=pltpu.CompilerParams(use_tc_tiling_on_sc=False),
        name="gin_edge_agg",
    )
    return fn(h, src_r, dst_r)


def _bn(a, g, b):
    mu = jnp.mean(a, axis=0, keepdims=True)
    d = a - mu
    var = jnp.mean(d * d, axis=0, keepdims=True)
    return g * (d * lax.rsqrt(var + EPS)) + b


def _mlp_core(h_ref, p_ref, w1, b1, g1, be1, w2, b2, g2, be2):
    h = h_ref[...]
    m = h + p_ref[0, :N, :] + p_ref[1, :N, :]
    a = jnp.dot(m, w1[...], preferred_element_type=jnp.float32) + b1[...]
    a = jnp.maximum(_bn(a, g1[...], be1[...]), 0.0)
    a = jnp.dot(a, w2[...], preferred_element_type=jnp.float32) + b2[...]
    a = jnp.maximum(_bn(a, g2[...], be2[...]), 0.0)
    return h, a


def _mlp_first_body(h_ref, p_ref, w1, b1, g1, be1, w2, b2, g2, be2,
                    wfc0, wfc1, hout_ref, zout_ref):
    h, a = _mlp_core(h_ref, p_ref, w1, b1, g1, be1, w2, b2, g2, be2)
    hout_ref[...] = a
    zout_ref[...] = (jnp.dot(h, wfc0[...], preferred_element_type=jnp.float32)
                     + jnp.dot(a, wfc1[...], preferred_element_type=jnp.float32))


def _mlp_mid_body(h_ref, p_ref, w1, b1, g1, be1, w2, b2, g2, be2,
                  wfc, zin_ref, hout_ref, zout_ref):
    _, a = _mlp_core(h_ref, p_ref, w1, b1, g1, be1, w2, b2, g2, be2)
    hout_ref[...] = a
    zout_ref[...] = zin_ref[...] + jnp.dot(
        a, wfc[...], preferred_element_type=jnp.float32)


def _mlp_last_body(h_ref, p_ref, w1, b1, g1, be1, w2, b2, g2, be2,
                   wfc, zin_ref, batch_ref, bias_ref, out_ref):
    _, a = _mlp_core(h_ref, p_ref, w1, b1, g1, be1, w2, b2, g2, be2)
    z = zin_ref[...] + jnp.dot(a, wfc[...], preferred_element_type=jnp.float32)
    row = lax.broadcasted_iota(jnp.int32, (G, N), 0)
    sel = (row == jnp.broadcast_to(batch_ref[...], (G, N))).astype(jnp.float32)
    bias = jnp.sum(bias_ref[...], axis=0, keepdims=True)
    out_ref[...] = jnp.dot(sel, z, preferred_element_type=jnp.float32) + bias


def kernel(x, edge_index, batch, params):
    src_r = edge_index[0].reshape(NW, CHUNKS, CSZ)
    dst_r = edge_index[1].reshape(NW, CHUNKS, CSZ)
    batch2 = batch.reshape(1, N)
    fcs = params['fcs']
    bias_stack = jnp.stack([fcs[i]['b'] for i in range(L + 1)])

    def cp(i):
        p = params['conv%d' % i]
        return (p['W1'], p['b1'].reshape(1, H), p['g1'].reshape(1, H),
                p['be1'].reshape(1, H), p['W2'], p['b2'].reshape(1, H),
                p['g'].reshape(1, H), p['be'].reshape(1, H))

    hz_shape = [jax.ShapeDtypeStruct((N, H), jnp.float32),
                jax.ShapeDtypeStruct((N, OUT), jnp.float32)]

    p = _edge_agg(x, src_r, dst_r)
    h, z = pl.pallas_call(_mlp_first_body, out_shape=hz_shape)(
        x, p, *cp(0), fcs[0]['W'], fcs[1]['W'])

    for i in (1, 2):
        p = _edge_agg(h, src_r, dst_r)
        h, z = pl.pallas_call(_mlp_mid_body, out_shape=hz_shape)(
            h, p, *cp(i), fcs[i + 1]['W'], z)

    p = _edge_agg(h, src_r, dst_r)
    out = pl.pallas_call(
        _mlp_last_body,
        out_shape=jax.ShapeDtypeStruct((G, OUT), jnp.float32))(
        h, p, *cp(3), fcs[4]['W'], z, batch2, bias_stack)
    return out


# speedup vs baseline: 1.4610x; 1.0045x over previous
import functools

import jax
import jax.numpy as jnp
from jax import lax
from jax.experimental import pallas as pl
from jax.experimental.pallas import tpu as pltpu
from jax.experimental.pallas import tpu_sc as plsc

N = 10000
E = 320000
D = 128
H = 128
OUT = 64
G = 128
L = 4
EPS = 1e-5

NC = 2
NS = 16
NW = NC * NS
EW = E // NW
CSZ = 100
CHUNKS = EW // CSZ
HALF = CHUNKS // 2
NPAD = 10000
RPT = NPAD // NS
ZROWS = 25


def _agg_body(h_hbm, src_hbm, dst_hbm, out_hbm,
              idx_s, idx_d, rows, acc, gsem, ssem):
    c = lax.axis_index("c")
    s = lax.axis_index("s")
    wid = s * NC + c

    def _zb(k, carry):
        rows[0, k // 8, pl.ds((k % 8) * 16, 16)] = jnp.zeros((16,), jnp.float32)
        return carry
    lax.fori_loop(0, ZROWS * 8, _zb, None)

    def _zc(t, carry):
        pltpu.async_copy(rows.at[0, pl.ds(0, ZROWS)],
                         acc.at[pl.ds(s * RPT + t * ZROWS, ZROWS)],
                         gsem.at[0])
        return carry
    lax.fori_loop(0, RPT // ZROWS, _zc, None)

    def _zw(t, carry):
        pltpu.make_async_copy(rows.at[0, pl.ds(0, ZROWS)],
                              acc.at[pl.ds(0, ZROWS)], gsem.at[0]).wait()
        return carry
    lax.fori_loop(0, RPT // ZROWS, _zw, None)
    plsc.subcore_barrier()

    def _g(j, b):
        pltpu.async_copy(h_hbm.at[idx_s.at[j]], rows.at[b], gsem.at[b])

    def _gw(j, b):
        pltpu.make_async_copy(h_hbm.at[idx_s.at[j]], rows.at[b],
                              gsem.at[b]).wait()

    def _s(j, b):
        pltpu.async_copy(rows.at[b], acc.at[idx_d.at[j]], ssem.at[b],
                         add=True)

    def _sw(j, b):
        pltpu.make_async_copy(rows.at[b], acc.at[idx_d.at[j]],
                              ssem.at[b]).wait()

    for p in range(2):
        pltpu.sync_copy(src_hbm.at[wid, pl.ds(p * HALF, HALF)], idx_s)
        pltpu.sync_copy(dst_hbm.at[wid, pl.ds(p * HALF, HALF)], idx_d)

        _g(0, 0)
        _g(1, 1)
        _gw(0, 0); _s(0, 0); _g(2, 2)

        def _body(q, carry):
            for k in range(3):
                j = 3 * q + 1 + k
                b = (1 + k) % 3
                b2 = k % 3
                _gw(j, b)
                _s(j, b)
                _sw(j - 1, b2)
                _g(j + 2, b2)
            return carry
        lax.fori_loop(0, 15, _body, None)

        _gw(46, 1); _s(46, 1); _sw(45, 0); _g(48, 0)
        _gw(47, 2); _s(47, 2); _sw(46, 1); _g(49, 1)
        _gw(48, 0); _s(48, 0); _sw(47, 2)
        _gw(49, 1); _s(49, 1); _sw(48, 0)
        _sw(49, 1)

    plsc.subcore_barrier()
    pltpu.sync_copy(acc.at[pl.ds(s * RPT, RPT)],
                    out_hbm.at[c, pl.ds(s * RPT, RPT)])


def _edge_agg(h, src_r, dst_r):
    mesh = plsc.VectorSubcoreMesh(core_axis_name="c", subcore_axis_name="s")
    fn = pl.kernel(
        _agg_body,
        out_type=jax.ShapeDtypeStruct((NC, NPAD, H), jnp.float32),
        mesh=mesh,
        scratch_types=[
            pltpu.VMEM((HALF, CSZ), jnp.int32),
            pltpu.VMEM((HALF, CSZ), jnp.int32),
            pltpu.VMEM((3, CSZ, H), jnp.float32),
            pltpu.VMEM_SHARED((NPAD, H), jnp.float32),
            pltpu.SemaphoreType.DMA((3,)),
            pltpu.SemaphoreType.DMA((3,)),
        ],
        compiler_params=pltpu.CompilerParams(use_tc_tiling_on_sc=False),
        name="gin_edge_agg",
    )
    return fn(h, src_r, dst_r)


def _bn(a, g, b):
    mu = jnp.mean(a, axis=0, keepdims=True)
    d = a - mu
    var = jnp.mean(d * d, axis=0, keepdims=True)
    return g * (d * lax.rsqrt(var + EPS)) + b


def _mlp_core(h_ref, p_ref, w1, b1, g1, be1, w2, b2, g2, be2):
    h = h_ref[...]
    m = h + p_ref[0, :N, :] + p_ref[1, :N, :]
    a = jnp.dot(m, w1[...], preferred_element_type=jnp.float32) + b1[...]
    a = jnp.maximum(_bn(a, g1[...], be1[...]), 0.0)
    a = jnp.dot(a, w2[...], preferred_element_type=jnp.float32) + b2[...]
    a = jnp.maximum(_bn(a, g2[...], be2[...]), 0.0)
    return h, a


def _mlp_first_body(h_ref, p_ref, w1, b1, g1, be1, w2, b2, g2, be2,
                    wfc0, wfc1, hout_ref, zout_ref):
    h, a = _mlp_core(h_ref, p_ref, w1, b1, g1, be1, w2, b2, g2, be2)
    hout_ref[...] = a
    zout_ref[...] = (jnp.dot(h, wfc0[...], preferred_element_type=jnp.float32)
                     + jnp.dot(a, wfc1[...], preferred_element_type=jnp.float32))


def _mlp_mid_body(h_ref, p_ref, w1, b1, g1, be1, w2, b2, g2, be2,
                  wfc, zin_ref, hout_ref, zout_ref):
    _, a = _mlp_core(h_ref, p_ref, w1, b1, g1, be1, w2, b2, g2, be2)
    hout_ref[...] = a
    zout_ref[...] = zin_ref[...] + jnp.dot(
        a, wfc[...], preferred_element_type=jnp.float32)


def _mlp_last_body(h_ref, p_ref, w1, b1, g1, be1, w2, b2, g2, be2,
                   wfc, zin_ref, batch_ref, bias_ref, out_ref):
    _, a = _mlp_core(h_ref, p_ref, w1, b1, g1, be1, w2, b2, g2, be2)
    z = zin_ref[...] + jnp.dot(a, wfc[...], preferred_element_type=jnp.float32)
    row = lax.broadcasted_iota(jnp.int32, (G, N), 0)
    sel = (row == jnp.broadcast_to(batch_ref[...], (G, N))).astype(jnp.float32)
    bias = jnp.sum(bias_ref[...], axis=0, keepdims=True)
    out_ref[...] = jnp.dot(sel, z, preferred_element_type=jnp.float32) + bias


def kernel(x, edge_index, batch, params):
    src_r = edge_index[0].reshape(NW, CHUNKS, CSZ)
    dst_r = edge_index[1].reshape(NW, CHUNKS, CSZ)
    batch2 = batch.reshape(1, N)
    fcs = params['fcs']
    bias_stack = jnp.stack([fcs[i]['b'] for i in range(L + 1)])

    def cp(i):
        p = params['conv%d' % i]
        return (p['W1'], p['b1'].reshape(1, H), p['g1'].reshape(1, H),
                p['be1'].reshape(1, H), p['W2'], p['b2'].reshape(1, H),
                p['g'].reshape(1, H), p['be'].reshape(1, H))

    hz_shape = [jax.ShapeDtypeStruct((N, H), jnp.float32),
                jax.ShapeDtypeStruct((N, OUT), jnp.float32)]

    p = _edge_agg(x, src_r, dst_r)
    h, z = pl.pallas_call(_mlp_first_body, out_shape=hz_shape)(
        x, p, *cp(0), fcs[0]['W'], fcs[1]['W'])

    for i in (1, 2):
        p = _edge_agg(h, src_r, dst_r)
        h, z = pl.pallas_call(_mlp_mid_body, out_shape=hz_shape)(
            h, p, *cp(i), fcs[i + 1]['W'], z)

    p = _edge_agg(h, src_r, dst_r)
    out = pl.pallas_call(
        _mlp_last_body,
        out_shape=jax.ShapeDtypeStruct((G, OUT), jnp.float32))(
        h, p, *cp(3), fcs[4]['W'], z, batch2, bias_stack)
    return out
